# baseline (speedup 1.0000x reference)
import functools

import jax
import jax.numpy as jnp
from jax import lax
from jax.experimental import pallas as pl
from jax.experimental.pallas import tpu as pltpu
from jax.experimental.pallas import tpu_sc as plsc

B = 4096
D = 64
Z = 10
N_DOC = 1000000
HALF = N_DOC // 2
NC = 2
NS = 16
NW = NC * NS
BPW = B // NW
EPW = BPW * (1 + Z)
CHUNK = 64
CHUNKS = EPW // CHUNK
POS_CHUNKS = BPW // CHUNK
SUBG = CHUNK // 16
PAIRS = CHUNKS // 2


def _sc_dots_build():
    mesh = plsc.VectorSubcoreMesh(core_axis_name="c", subcore_axis_name="s")

    @functools.partial(
        pl.kernel,
        out_type=jax.ShapeDtypeStruct((NW, EPW), jnp.float32),
        mesh=mesh,
        compiler_params=pltpu.CompilerParams(
            use_tc_tiling_on_sc=False, needs_layout_passes=False),
        scratch_types=[
            pltpu.VMEM((CHUNKS + 2, CHUNK), jnp.int32),
            pltpu.VMEM((CHUNKS + 2, CHUNK), jnp.int32),
            pltpu.VMEM((CHUNKS + 2, CHUNK), jnp.int32),
            pltpu.VMEM((BPW, D), jnp.float32),
            pltpu.VMEM((2 * CHUNK, D), jnp.float32),
            pltpu.VMEM((2 * CHUNK, D), jnp.float32),
            pltpu.VMEM((EPW,), jnp.float32),
            pltpu.SemaphoreType.DMA,
            pltpu.SemaphoreType.DMA,
        ],
    )
    def sc_dots(ilo_hbm, ihi_hbm, rsel_hbm, q_hbm, tlo_hbm, thi_hbm, out_hbm,
                ilo_v, ihi_v, rsel_v, q_v, gbuf0, gbuf1, dots_v, sem0, sem1):
        w = lax.axis_index("s") * NC + lax.axis_index("c")
        pltpu.sync_copy(ilo_hbm.at[w], ilo_v)
        pltpu.sync_copy(ihi_hbm.at[w], ihi_v)
        pltpu.sync_copy(rsel_hbm.at[w], rsel_v)
        pltpu.sync_copy(q_hbm.at[pl.ds(w * BPW, BPW)], q_v)

        lane = lax.broadcasted_iota(jnp.int32, (16,), 0)

        def fire(c, gbuf, sem):
            pltpu.async_copy(tlo_hbm.at[ilo_v.at[c]],
                             gbuf.at[pl.ds(0, CHUNK)], sem)
            pltpu.async_copy(thi_hbm.at[ihi_v.at[c]],
                             gbuf.at[pl.ds(CHUNK, CHUNK)], sem)

        def drain(gbuf, sem):
            pltpu.make_async_copy(tlo_hbm.at[pl.ds(0, 2 * CHUNK)], gbuf,
                                  sem).wait()

        def compute(c, gbuf):
            def subgroup(s, _):
                rsel = rsel_v[c, pl.ds(s * 16, 16)]
                r_vec = lane + s * 16 + rsel
                base = c * CHUNK + s * 16 + lane
                neg_b = (base - POS_CHUNKS * CHUNK) // Z
                b_vec = jnp.where(jnp.full((16,), c < POS_CHUNKS), base,
                                  neg_b)

                acc = jnp.zeros((16,), jnp.float32)
                for d in range(D):
                    d_vec = jnp.full((16,), d, dtype=jnp.int32)
                    doc = plsc.load_gather(gbuf, [r_vec, d_vec])
                    qv = plsc.load_gather(q_v, [b_vec, d_vec])
                    acc = acc + doc * qv
                dots_v[pl.ds(c * CHUNK + s * 16, 16)] = acc
                return 0

            lax.fori_loop(0, SUBG, subgroup, 0)

        fire(0, gbuf0, sem0)
        fire(1, gbuf1, sem1)

        def pair(i, _):
            c0 = 2 * i
            drain(gbuf0, sem0)
            compute(c0, gbuf0)
            fire(c0 + 2, gbuf0, sem0)
            drain(gbuf1, sem1)
            compute(c0 + 1, gbuf1)
            fire(c0 + 3, gbuf1, sem1)
            return 0

        lax.fori_loop(0, PAIRS, pair, 0)
        drain(gbuf0, sem0)
        drain(gbuf1, sem1)

        pltpu.sync_copy(dots_v, out_hbm.at[w])

    return sc_dots


_sc_dots = _sc_dots_build()


def _tc_score_body(pos_ref, neg_ref, o_ref):
    pos_dot = pos_ref[...]
    neg_dot = neg_ref[...]
    pos_repr = 1.0 / (1.0 + jnp.exp(-pos_dot))
    neg_repr = 1.0 / (1.0 + jnp.exp(-neg_dot))
    positive_term = jnp.log(pos_repr)
    negative_term = jnp.sum(jnp.log(1.0 - neg_repr + 1e-40), axis=1,
                            keepdims=True)
    zf = float(Z)
    o_ref[...] = (zf + 1.0) / (2.0 * zf) * (zf * positive_term + negative_term)


def kernel(query, document, doc_emb, neg_sample):
    doc_i = document.astype(jnp.int32).reshape(NW, BPW)
    neg_i = neg_sample.astype(jnp.int32).reshape(NW, BPW * Z)
    pad = jnp.zeros((NW, 2 * CHUNK), jnp.int32)
    idx_all = jnp.concatenate([doc_i, neg_i, pad], axis=1).reshape(
        NW, CHUNKS + 2, CHUNK)
    is_hi = idx_all >= HALF
    idx_lo = jnp.where(is_hi, 0, idx_all)
    idx_hi = jnp.where(is_hi, idx_all - HALF, 0)
    rsel = jnp.where(is_hi, CHUNK, 0).astype(jnp.int32)
    tlo = doc_emb[:HALF]
    thi = doc_emb[HALF:]
    dots = _sc_dots(idx_lo, idx_hi, rsel, query, tlo, thi)
    pos_dot = dots[:, :BPW].reshape(B, 1)
    neg_dot = dots[:, BPW:].reshape(B, Z)
    out = pl.pallas_call(
        _tc_score_body,
        out_shape=jax.ShapeDtypeStruct((B, 1), jnp.float32),
    )(pos_dot, neg_dot)
    return out.reshape(B)


# speedup vs baseline: 1.6442x; 1.6442x over previous
import functools

import jax
import jax.numpy as jnp
from jax import lax
from jax.experimental import pallas as pl
from jax.experimental.pallas import tpu as pltpu
from jax.experimental.pallas import tpu_sc as plsc

B = 4096
D = 64
Z = 10
N_DOC = 1000000
NC = 2
NS = 16
NW = NC * NS
BPW = B // NW
EPW = BPW * (1 + Z)
CHUNK = 64
CHUNKS = EPW // CHUNK
POS_CHUNKS = BPW // CHUNK
SUBG = CHUNK // 16
PAIRS = CHUNKS // 2


def _sc_dots_build():
    mesh = plsc.VectorSubcoreMesh(core_axis_name="c", subcore_axis_name="s")

    @functools.partial(
        pl.kernel,
        out_type=jax.ShapeDtypeStruct((NW, EPW), jnp.float32),
        mesh=mesh,
        compiler_params=pltpu.CompilerParams(needs_layout_passes=False),
        scratch_types=[
            pltpu.VMEM((CHUNKS + 2, CHUNK), jnp.int32),
            pltpu.VMEM((CHUNKS + 2, CHUNK), jnp.int32),
            pltpu.VMEM((BPW, D), jnp.float32),
            pltpu.VMEM((CHUNK, 2 * D), jnp.float32),
            pltpu.VMEM((CHUNK, 2 * D), jnp.float32),
            pltpu.VMEM((EPW,), jnp.float32),
            pltpu.SemaphoreType.DMA,
            pltpu.SemaphoreType.DMA,
        ],
    )
    def sc_dots(ig_hbm, jsel_hbm, q_hbm, table_hbm, out_hbm,
                ig_v, jsel_v, q_v, gbuf0, gbuf1, dots_v, sem0, sem1):
        w = lax.axis_index("s") * NC + lax.axis_index("c")
        pltpu.sync_copy(ig_hbm.at[w], ig_v)
        pltpu.sync_copy(jsel_hbm.at[w], jsel_v)
        pltpu.sync_copy(q_hbm.at[pl.ds(w * BPW, BPW)], q_v)

        lane = lax.broadcasted_iota(jnp.int32, (16,), 0)

        def fire(c, gbuf, sem):
            pltpu.async_copy(table_hbm.at[ig_v.at[c]], gbuf, sem)

        def drain(gbuf, sem):
            pltpu.make_async_copy(table_hbm.at[pl.ds(0, CHUNK)], gbuf,
                                  sem).wait()

        def compute(c, gbuf):
            def subgroup(s, _):
                jsel = jsel_v[c, pl.ds(s * 16, 16)]
                r_vec = lane + s * 16
                base = c * CHUNK + s * 16 + lane
                neg_b = (base - POS_CHUNKS * CHUNK) // Z
                b_vec = jnp.where(jnp.full((16,), c < POS_CHUNKS), base,
                                  neg_b)

                acc = jnp.zeros((16,), jnp.float32)
                for d in range(D):
                    d_vec = jnp.full((16,), d, dtype=jnp.int32)
                    doc = plsc.load_gather(gbuf, [r_vec, jsel + d_vec])
                    qv = plsc.load_gather(q_v, [b_vec, d_vec])
                    acc = acc + doc * qv
                dots_v[pl.ds(c * CHUNK + s * 16, 16)] = acc
                return 0

            lax.fori_loop(0, SUBG, subgroup, 0)

        fire(0, gbuf0, sem0)
        fire(1, gbuf1, sem1)

        def pair(i, _):
            c0 = 2 * i
            drain(gbuf0, sem0)
            compute(c0, gbuf0)
            fire(c0 + 2, gbuf0, sem0)
            drain(gbuf1, sem1)
            compute(c0 + 1, gbuf1)
            fire(c0 + 3, gbuf1, sem1)
            return 0

        lax.fori_loop(0, PAIRS, pair, 0)
        drain(gbuf0, sem0)
        drain(gbuf1, sem1)

        pltpu.sync_copy(dots_v, out_hbm.at[w])

    return sc_dots


_sc_dots = _sc_dots_build()


def _tc_score_body(pos_ref, neg_ref, o_ref):
    pos_dot = pos_ref[...]
    neg_dot = neg_ref[...]
    pos_repr = 1.0 / (1.0 + jnp.exp(-pos_dot))
    neg_repr = 1.0 / (1.0 + jnp.exp(-neg_dot))
    positive_term = jnp.log(pos_repr)
    negative_term = jnp.sum(jnp.log(1.0 - neg_repr + 1e-40), axis=1,
                            keepdims=True)
    zf = float(Z)
    o_ref[...] = (zf + 1.0) / (2.0 * zf) * (zf * positive_term + negative_term)


def kernel(query, document, doc_emb, neg_sample):
    doc_i = document.astype(jnp.int32).reshape(NW, BPW)
    neg_i = neg_sample.astype(jnp.int32).reshape(NW, BPW * Z)
    pad = jnp.zeros((NW, 2 * CHUNK), jnp.int32)
    idx_all = jnp.concatenate([doc_i, neg_i, pad], axis=1).reshape(
        NW, CHUNKS + 2, CHUNK)
    idx_g = idx_all >> 1
    jsel = (idx_all & 1) * D
    table2 = doc_emb.reshape(N_DOC // 2, 2 * D)
    dots = _sc_dots(idx_g, jsel, query, table2)
    pos_dot = dots[:, :BPW].reshape(B, 1)
    neg_dot = dots[:, BPW:].reshape(B, Z)
    out = pl.pallas_call(
        _tc_score_body,
        out_shape=jax.ShapeDtypeStruct((B, 1), jnp.float32),
    )(pos_dot, neg_dot)
    return out.reshape(B)


# speedup vs baseline: 1.9214x; 1.1685x over previous
import functools

import jax
import jax.numpy as jnp
from jax import lax
from jax.experimental import pallas as pl
from jax.experimental.pallas import tpu as pltpu
from jax.experimental.pallas import tpu_sc as plsc

B = 4096
D = 64
Z = 10
N_DOC = 1000000
GW = 512
NC = 2
NS = 16
NW = NC * NS
BPW = B // NW
EPW = BPW * (1 + Z)
CHUNK = 32
CHUNKS = EPW // CHUNK
POS_CHUNKS = BPW // CHUNK
SUBG = CHUNK // 16
RING = 3
DU = 16


def _sc_dots_build():
    mesh = plsc.VectorSubcoreMesh(core_axis_name="c", subcore_axis_name="s")

    @functools.partial(
        pl.kernel,
        out_type=jax.ShapeDtypeStruct((NW, EPW), jnp.float32),
        mesh=mesh,
        compiler_params=pltpu.CompilerParams(needs_layout_passes=False),
        scratch_types=[
            pltpu.VMEM((CHUNKS, CHUNK), jnp.int32),
            pltpu.VMEM((CHUNKS, CHUNK), jnp.int32),
            pltpu.VMEM((BPW, D), jnp.float32),
            pltpu.VMEM((CHUNK, GW), jnp.float32),
            pltpu.VMEM((CHUNK, GW), jnp.float32),
            pltpu.VMEM((CHUNK, GW), jnp.float32),
            pltpu.VMEM((EPW,), jnp.float32),
            pltpu.SemaphoreType.DMA,
            pltpu.SemaphoreType.DMA,
            pltpu.SemaphoreType.DMA,
        ],
    )
    def sc_dots(ig_hbm, jsel_hbm, q_hbm, table_hbm, out_hbm,
                ig_v, jsel_v, q_v, gb0, gb1, gb2, dots_v, sm0, sm1, sm2):
        w = lax.axis_index("s") * NC + lax.axis_index("c")
        pltpu.sync_copy(ig_hbm.at[w], ig_v)
        pltpu.sync_copy(jsel_hbm.at[w], jsel_v)
        pltpu.sync_copy(q_hbm.at[pl.ds(w * BPW, BPW)], q_v)

        gbufs = (gb0, gb1, gb2)
        sems = (sm0, sm1, sm2)
        lane = lax.broadcasted_iota(jnp.int32, (16,), 0)

        def fire(c, slot):
            pltpu.async_copy(table_hbm.at[ig_v.at[c]], gbufs[slot],
                             sems[slot])

        def drain(slot):
            pltpu.make_async_copy(table_hbm.at[pl.ds(0, CHUNK)],
                                  gbufs[slot], sems[slot]).wait()

        def compute(c, slot):
            gbuf = gbufs[slot]

            def subgroup(s, _):
                jsel = jsel_v[c, pl.ds(s * 16, 16)]
                r_vec = lane + s * 16
                base = c * CHUNK + s * 16 + lane
                neg_b = (base - POS_CHUNKS * CHUNK) // Z
                b_vec = jnp.where(jnp.full((16,), c < POS_CHUNKS), base,
                                  neg_b)

                def dblock(k, acc):
                    d0 = k * DU
                    for dd in range(DU):
                        d_vec = jnp.full((16,), dd, dtype=jnp.int32) + d0
                        doc = plsc.load_gather(gbuf, [r_vec, jsel + d_vec])
                        qv = plsc.load_gather(q_v, [b_vec, d_vec])
                        acc = acc + doc * qv
                    return acc

                acc = lax.fori_loop(0, D // DU, dblock,
                                    jnp.zeros((16,), jnp.float32))
                dots_v[pl.ds(c * CHUNK + s * 16, 16)] = acc
                return 0

            lax.fori_loop(0, SUBG, subgroup, 0)

        for c in range(RING):
            fire(c, c % RING)
        for c in range(CHUNKS):
            slot = c % RING
            drain(slot)
            compute(c, slot)
            if c + RING < CHUNKS:
                fire(c + RING, slot)

        pltpu.sync_copy(dots_v, out_hbm.at[w])

    return sc_dots


_sc_dots = _sc_dots_build()


def _tc_score_body(pos_ref, neg_ref, o_ref):
    pos_dot = pos_ref[...]
    neg_dot = neg_ref[...]
    pos_repr = 1.0 / (1.0 + jnp.exp(-pos_dot))
    neg_repr = 1.0 / (1.0 + jnp.exp(-neg_dot))
    positive_term = jnp.log(pos_repr)
    negative_term = jnp.sum(jnp.log(1.0 - neg_repr + 1e-40), axis=1,
                            keepdims=True)
    zf = float(Z)
    o_ref[...] = (zf + 1.0) / (2.0 * zf) * (zf * positive_term + negative_term)


def kernel(query, document, doc_emb, neg_sample):
    doc_i = document.astype(jnp.int32).reshape(NW, BPW)
    neg_i = neg_sample.astype(jnp.int32).reshape(NW, BPW * Z)
    idx_all = jnp.concatenate([doc_i, neg_i], axis=1).reshape(
        NW, CHUNKS, CHUNK)
    idx_g = idx_all >> 3
    jsel = (idx_all & 7) * D
    table8 = doc_emb.reshape(N_DOC // 8, GW)
    dots = _sc_dots(idx_g, jsel, query, table8)
    pos_dot = dots[:, :BPW].reshape(B, 1)
    neg_dot = dots[:, BPW:].reshape(B, Z)
    out = pl.pallas_call(
        _tc_score_body,
        out_shape=jax.ShapeDtypeStruct((B, 1), jnp.float32),
    )(pos_dot, neg_dot)
    return out.reshape(B)


# speedup vs baseline: 3.4010x; 1.7701x over previous
import functools

import jax
import jax.numpy as jnp
from jax import lax
from jax.experimental import pallas as pl
from jax.experimental.pallas import tpu as pltpu
from jax.experimental.pallas import tpu_sc as plsc

B = 4096
D = 64
Z = 10
N_DOC = 1000000
NC = 2
NS = 16
NW = NC * NS
BPW = B // NW
EPW = BPW * (1 + Z)
CHUNK = 32
CHUNKS = EPW // CHUNK
POS_CHUNKS = BPW // CHUNK
SUBG = CHUNK // 16
PAIRS = CHUNKS // 2
DU = 16


def _sc_dots_build():
    mesh = plsc.VectorSubcoreMesh(core_axis_name="c", subcore_axis_name="s")

    @functools.partial(
        pl.kernel,
        out_type=jax.ShapeDtypeStruct((NW, EPW), jnp.float32),
        mesh=mesh,
        compiler_params=pltpu.CompilerParams(needs_layout_passes=False),
        scratch_types=[
            pltpu.VMEM((CHUNKS + 2, CHUNK), jnp.int32),
            pltpu.VMEM((CHUNKS + 2, CHUNK), jnp.int32),
            pltpu.VMEM((BPW, D), jnp.float32),
            pltpu.VMEM((CHUNK, 8, D), jnp.float32),
            pltpu.VMEM((CHUNK, 8, D), jnp.float32),
            pltpu.VMEM((EPW,), jnp.float32),
            pltpu.SemaphoreType.DMA,
            pltpu.SemaphoreType.DMA,
        ],
    )
    def sc_dots(ig_hbm, jsel_hbm, q_hbm, table_hbm, out_hbm,
                ig_v, jsel_v, q_v, gbuf0, gbuf1, dots_v, sem0, sem1):
        w = lax.axis_index("s") * NC + lax.axis_index("c")
        pltpu.sync_copy(ig_hbm.at[w], ig_v)
        pltpu.sync_copy(jsel_hbm.at[w], jsel_v)
        pltpu.sync_copy(q_hbm.at[pl.ds(w * BPW, BPW)], q_v)

        lane = lax.broadcasted_iota(jnp.int32, (16,), 0)

        def fire(c, gbuf, sem):
            for s in range(SUBG):
                gvec = ig_v[c, pl.ds(s * 16, 16)]
                for t in range(16):
                    pltpu.async_copy(table_hbm.at[gvec[t]],
                                     gbuf.at[s * 16 + t], sem)

        def drain(gbuf, sem):
            pltpu.make_async_copy(table_hbm.at[pl.ds(0, CHUNK)], gbuf,
                                  sem).wait()

        def compute(c, gbuf):
            def subgroup(s, _):
                jsel = jsel_v[c, pl.ds(s * 16, 16)]
                r_vec = lane + s * 16
                base = c * CHUNK + s * 16 + lane
                neg_b = (base - POS_CHUNKS * CHUNK) // Z
                b_vec = jnp.where(jnp.full((16,), c < POS_CHUNKS), base,
                                  neg_b)
                j8 = jsel >> 6

                def dblock(k, acc):
                    d0 = k * DU
                    for dd in range(DU):
                        d_vec = jnp.full((16,), dd, dtype=jnp.int32) + d0
                        doc = plsc.load_gather(gbuf, [r_vec, j8, d_vec])
                        qv = plsc.load_gather(q_v, [b_vec, d_vec])
                        acc = acc + doc * qv
                    return acc

                acc = lax.fori_loop(0, D // DU, dblock,
                                    jnp.zeros((16,), jnp.float32))
                dots_v[pl.ds(c * CHUNK + s * 16, 16)] = acc
                return 0

            lax.fori_loop(0, SUBG, subgroup, 0)

        fire(0, gbuf0, sem0)
        fire(1, gbuf1, sem1)

        def pair(i, _):
            c0 = 2 * i
            drain(gbuf0, sem0)
            compute(c0, gbuf0)
            fire(c0 + 2, gbuf0, sem0)
            drain(gbuf1, sem1)
            compute(c0 + 1, gbuf1)
            fire(c0 + 3, gbuf1, sem1)
            return 0

        lax.fori_loop(0, PAIRS, pair, 0)
        drain(gbuf0, sem0)
        drain(gbuf1, sem1)

        pltpu.sync_copy(dots_v, out_hbm.at[w])

    return sc_dots


_sc_dots = _sc_dots_build()


def _tc_score_body(pos_ref, neg_ref, o_ref):
    pos_dot = pos_ref[...]
    neg_dot = neg_ref[...]
    pos_repr = 1.0 / (1.0 + jnp.exp(-pos_dot))
    neg_repr = 1.0 / (1.0 + jnp.exp(-neg_dot))
    positive_term = jnp.log(pos_repr)
    negative_term = jnp.sum(jnp.log(1.0 - neg_repr + 1e-40), axis=1,
                            keepdims=True)
    zf = float(Z)
    o_ref[...] = (zf + 1.0) / (2.0 * zf) * (zf * positive_term + negative_term)


def kernel(query, document, doc_emb, neg_sample):
    doc_i = document.astype(jnp.int32).reshape(NW, BPW)
    neg_i = neg_sample.astype(jnp.int32).reshape(NW, BPW * Z)
    pad = jnp.zeros((NW, 2 * CHUNK), jnp.int32)
    idx_all = jnp.concatenate([doc_i, neg_i, pad], axis=1).reshape(
        NW, CHUNKS + 2, CHUNK)
    idx_g = idx_all >> 3
    jsel = (idx_all & 7) * D
    table3 = doc_emb.reshape(N_DOC // 8, 8, D)
    dots = _sc_dots(idx_g, jsel, query, table3)
    pos_dot = dots[:, :BPW].reshape(B, 1)
    neg_dot = dots[:, BPW:].reshape(B, Z)
    out = pl.pallas_call(
        _tc_score_body,
        out_shape=jax.ShapeDtypeStruct((B, 1), jnp.float32),
    )(pos_dot, neg_dot)
    return out.reshape(B)
